# baseline (speedup 1.0000x reference)
import functools

import jax
import jax.numpy as jnp
from jax import lax
from jax.experimental import pallas as pl
from jax.experimental.pallas import tpu as pltpu
from jax.experimental.pallas import tpu_sc as plsc

_GROUPS_PER_BATCH = 8
_SC_ROWS = 8192
_TC_RING = 16


def _sc_gather(idx2, table, b_per_w, nc, ns):
    nw = nc * ns
    n_groups = b_per_w // 16
    n_batches = n_groups // _GROUPS_PER_BATCH
    rows_per_batch = _GROUPS_PER_BATCH * 16
    D = table.shape[1]

    mesh = plsc.VectorSubcoreMesh(core_axis_name="c", subcore_axis_name="s")

    @functools.partial(
        pl.kernel,
        mesh=mesh,
        out_type=jax.ShapeDtypeStruct((nw * b_per_w, D), jnp.float32),
        scratch_types=[
            pltpu.VMEM((b_per_w,), jnp.int32),
            pltpu.VMEM((b_per_w, D), jnp.float32),
            pltpu.SemaphoreType.DMA,
        ],
    )
    def body(idx_hbm, table_hbm, out_hbm, idx_v, rows_v, sem):
        wid = lax.axis_index("s") * nc + lax.axis_index("c")
        base = wid * b_per_w
        pltpu.sync_copy(idx_hbm.at[wid], idx_v)

        def fire_group(g, carry):
            vec = idx_v[pl.ds(g * 16, 16)]
            for l in range(16):
                r = vec[l]
                pltpu.async_copy(table_hbm.at[pl.ds(r, 1)],
                                 rows_v.at[pl.ds(g * 16 + l, 1)], sem)
            return carry

        for f in range(n_batches):
            lax.fori_loop(f * _GROUPS_PER_BATCH, (f + 1) * _GROUPS_PER_BATCH,
                          fire_group, 0)
            pltpu.make_async_copy(
                table_hbm.at[pl.ds(0, rows_per_batch)],
                rows_v.at[pl.ds(f * rows_per_batch, rows_per_batch)],
                sem).wait()

        pltpu.sync_copy(rows_v, out_hbm.at[pl.ds(base, b_per_w)])

    return body(idx2, table)


def _tc_gather(idx_tc, table):
    n = idx_tc.shape[0]
    D = table.shape[1]

    def body(idx_smem, table_hbm, out_hbm, rows_v, sems, out_sem):
        def start(j):
            r = idx_smem[j]
            pltpu.make_async_copy(
                table_hbm.at[pl.ds(r, 1)],
                rows_v.at[pl.ds(j, 1)],
                sems.at[lax.rem(j, _TC_RING)]).start()

        def wait(j):
            pltpu.make_async_copy(
                table_hbm.at[pl.ds(0, 1)],
                rows_v.at[pl.ds(j, 1)],
                sems.at[lax.rem(j, _TC_RING)]).wait()

        def warm(j, c):
            start(j)
            return c

        lax.fori_loop(0, _TC_RING, warm, 0)

        def step(j, c):
            start(j)
            wait(j - _TC_RING)
            return c

        lax.fori_loop(_TC_RING, n, step, 0)

        def drain(j, c):
            wait(j)
            return c

        lax.fori_loop(n - _TC_RING, n, drain, 0)

        cp = pltpu.make_async_copy(rows_v, out_hbm, out_sem)
        cp.start()
        cp.wait()

    return pl.pallas_call(
        body,
        out_shape=jax.ShapeDtypeStruct((n, D), jnp.float32),
        in_specs=[
            pl.BlockSpec(memory_space=pltpu.SMEM),
            pl.BlockSpec(memory_space=pl.ANY),
        ],
        out_specs=pl.BlockSpec(memory_space=pl.ANY),
        scratch_shapes=[
            pltpu.VMEM((n, D), jnp.float32),
            pltpu.SemaphoreType.DMA((_TC_RING,)),
            pltpu.SemaphoreType.DMA,
        ],
    )(idx_tc, table)


def kernel(node_idx, table):
    B = node_idx.shape[0]
    info = plsc.get_sparse_core_info()
    nc, ns = info.num_cores, info.num_subcores
    nw = nc * ns
    b_per_w = _SC_ROWS // nw

    idx = node_idx.astype(jnp.int32)
    idx_sc = idx[:_SC_ROWS].reshape(nw, b_per_w)
    idx_tc = idx[_SC_ROWS:]

    out_sc = _sc_gather(idx_sc, table, b_per_w, nc, ns)
    out_tc = _tc_gather(idx_tc, table)
    return jnp.concatenate([out_sc, out_tc], axis=0)


# speedup vs baseline: 1.8652x; 1.8652x over previous
import functools

import jax
import jax.numpy as jnp
from jax import lax
from jax.experimental import pallas as pl
from jax.experimental.pallas import tpu as pltpu
from jax.experimental.pallas import tpu_sc as plsc

_NSEM = 4
_GROUPS_PER_BATCH = 8


def kernel(node_idx, table):
    B = node_idx.shape[0]
    V, D = table.shape
    info = plsc.get_sparse_core_info()
    NC, NS = info.num_cores, info.num_subcores
    NW = NC * NS
    b_per_w = B // NW
    n_groups = b_per_w // 16
    n_batches = n_groups // _GROUPS_PER_BATCH
    rows_per_batch = _GROUPS_PER_BATCH * 16
    rows_per_sem = rows_per_batch // _NSEM

    idx2 = node_idx.astype(jnp.int32).reshape(NW, b_per_w)

    mesh = plsc.VectorSubcoreMesh(core_axis_name="c", subcore_axis_name="s")

    @functools.partial(
        pl.kernel,
        mesh=mesh,
        out_type=jax.ShapeDtypeStruct((B, D), jnp.float32),
        scratch_types=[
            pltpu.VMEM((b_per_w,), jnp.int32),
            pltpu.VMEM((b_per_w, D), jnp.float32),
            pltpu.SemaphoreType.DMA((_NSEM,)),
        ],
    )
    def body(idx_hbm, table_hbm, out_hbm, idx_v, rows_v, sems):
        wid = lax.axis_index("s") * NC + lax.axis_index("c")
        base = wid * b_per_w
        pltpu.sync_copy(idx_hbm.at[wid], idx_v)

        def fire_group(g, carry):
            vec = idx_v[pl.ds(g * 16, 16)]
            for l in range(16):
                r = vec[l]
                pltpu.async_copy(table_hbm.at[pl.ds(r, 1)],
                                 rows_v.at[pl.ds(g * 16 + l, 1)],
                                 sems.at[l % _NSEM])
            return carry

        for f in range(n_batches):
            lax.fori_loop(f * _GROUPS_PER_BATCH, (f + 1) * _GROUPS_PER_BATCH,
                          fire_group, 0)
            for s in range(_NSEM):
                pltpu.make_async_copy(
                    table_hbm.at[pl.ds(0, rows_per_sem)],
                    rows_v.at[pl.ds(f * rows_per_batch + s * rows_per_sem,
                                    rows_per_sem)],
                    sems.at[s]).wait()

        pltpu.sync_copy(rows_v, out_hbm.at[pl.ds(base, b_per_w)])

    return body(idx2, table)
